# Initial kernel scaffold; baseline (speedup 1.0000x reference)
import jax
import jax.numpy as jnp
from jax.experimental import pallas as pl


def kernel(support, query, W, b):
    raise NotImplementedError("write your pallas kernel here")


# speedup vs baseline: 34.2133x; 34.2133x over previous
import functools

import jax
import jax.numpy as jnp
from jax import lax
from jax.experimental import pallas as pl
from jax.experimental.pallas import tpu as pltpu

_K_CL = 5
_K_PAD = 8
_MAX_ITER = 100


def _encoder_body(x_ref, w_ref, b_ref, o_ref):
    o_ref[...] = (
        jnp.dot(x_ref[...], w_ref[...], preferred_element_type=jnp.float32)
        + b_ref[...]
    )


def _encoder(x, W, b, block_rows=1024):
    n, d = x.shape
    while n % block_rows:
        block_rows //= 2
    grid = (n // block_rows,)
    return pl.pallas_call(
        _encoder_body,
        grid=grid,
        in_specs=[
            pl.BlockSpec((block_rows, d), lambda i: (i, 0)),
            pl.BlockSpec((d, d), lambda i: (0, 0)),
            pl.BlockSpec((1, d), lambda i: (0, 0)),
        ],
        out_specs=pl.BlockSpec((block_rows, d), lambda i: (i, 0)),
        out_shape=jax.ShapeDtypeStruct((n, d), jnp.float32),
    )(x, W, b.reshape(1, d))


def _gram_body(x_ref, k_ref):
    x = x_ref[0]
    k_ref[0] = lax.dot_general(
        x, x, (((1,), (1,)), ((), ())), preferred_element_type=jnp.float32
    )


def _gram(x):
    n_way, n, d = x.shape
    return pl.pallas_call(
        _gram_body,
        grid=(n_way,),
        in_specs=[pl.BlockSpec((1, n, d), lambda c: (c, 0, 0))],
        out_specs=pl.BlockSpec((1, n, n), lambda c: (c, 0, 0)),
        out_shape=jax.ShapeDtypeStruct((n_way, n, n), jnp.float32),
    )(x)


def _kmeans_body(k_ref, w0_ref, u_ref, w_ref, prev_ref):
    n_way, kp, n = w0_ref.shape
    w_ref[...] = w0_ref[...]
    prev_ref[...] = jnp.full((n_way, n), -1, jnp.int32)

    jidx = lax.broadcasted_iota(jnp.int32, (kp, n), 0)
    jbad = jidx >= _K_CL

    def one_class(c, changed):
        kc = k_ref[c]
        wc = w_ref[c]
        g = lax.dot_general(
            wc, kc, (((1,), (0,)), ((), ())), preferred_element_type=jnp.float32
        )
        c2 = jnp.sum(g * wc, axis=1, keepdims=True)
        score = c2 - 2.0 * g
        score = jnp.where(jbad, jnp.inf, score)
        smin = jnp.min(score, axis=0, keepdims=True)
        assign = jnp.min(
            jnp.where(score == smin, jidx, _K_PAD), axis=0, keepdims=True
        )
        onehot = (jidx == assign).astype(jnp.float32)
        counts = jnp.sum(onehot, axis=1, keepdims=True)
        wn = jnp.where(counts > 0.0, onehot / counts, wc)
        w_ref[c] = wn
        ch = jnp.any(assign[0] != prev_ref[c])
        prev_ref[c] = assign[0]
        return jnp.logical_or(changed, ch)

    def cond(carry):
        it, changed = carry
        return jnp.logical_and(it < _MAX_ITER, changed)

    def body(carry):
        it, _ = carry
        changed = lax.fori_loop(0, n_way, one_class, jnp.bool_(False))
        return it + 1, changed

    lax.while_loop(cond, body, (jnp.int32(0), jnp.bool_(True)))
    u_ref[...] = jnp.sum(w_ref[...], axis=1) * (1.0 / _K_CL)


def _kmeans(K, w0):
    n_way, kp, n = w0.shape
    return pl.pallas_call(
        _kmeans_body,
        in_specs=[
            pl.BlockSpec((n_way, n, n), lambda: (0, 0, 0)),
            pl.BlockSpec((n_way, kp, n), lambda: (0, 0, 0)),
        ],
        out_specs=pl.BlockSpec((n_way, n), lambda: (0, 0)),
        out_shape=jax.ShapeDtypeStruct((n_way, n), jnp.float32),
        scratch_shapes=[
            pltpu.VMEM((n_way, kp, n), jnp.float32),
            pltpu.VMEM((n_way, n), jnp.int32),
        ],
    )(K, w0)


def _proto_body(u_ref, x_ref, p_ref):
    p_ref[0] = jnp.dot(
        u_ref[0], x_ref[0], preferred_element_type=jnp.float32
    )


def _proto(u, x):
    n_way, n, d = x.shape
    out = pl.pallas_call(
        _proto_body,
        grid=(n_way,),
        in_specs=[
            pl.BlockSpec((1, 1, n), lambda c: (c, 0, 0)),
            pl.BlockSpec((1, n, d), lambda c: (c, 0, 0)),
        ],
        out_specs=pl.BlockSpec((1, 1, d), lambda c: (c, 0, 0)),
        out_shape=jax.ShapeDtypeStruct((n_way, 1, d), jnp.float32),
    )(u.reshape(n_way, 1, n), x)
    return out.reshape(n_way, d)


def _cdist_body(q_ref, p_ref, o_ref):
    q = q_ref[0]
    p = p_ref[...]
    q2 = jnp.sum(q * q, axis=1, keepdims=True)
    p2 = jnp.sum(p * p, axis=1, keepdims=True)
    qp = lax.dot_general(
        q, p, (((1,), (1,)), ((), ())), preferred_element_type=jnp.float32
    )
    d2 = q2 + p2.T - 2.0 * qp
    o_ref[0] = -jnp.sqrt(jnp.maximum(d2, 1e-12))


def _cdist_logits(q, p):
    n_way, nq, d = q.shape
    return pl.pallas_call(
        _cdist_body,
        grid=(n_way,),
        in_specs=[
            pl.BlockSpec((1, nq, d), lambda c: (c, 0, 0)),
            pl.BlockSpec((n_way, d), lambda c: (0, 0)),
        ],
        out_specs=pl.BlockSpec((1, nq, n_way), lambda c: (c, 0, 0)),
        out_shape=jax.ShapeDtypeStruct((n_way, nq, n_way), jnp.float32),
    )(q, p)


@jax.jit
def _pipeline(support, query, W, b):
    n_way, n_support, d = support.shape
    n_query = query.shape[1]

    kkey = jax.random.key(42)
    keys = jax.random.split(kkey, n_way)
    idx = jax.vmap(lambda k: jax.random.permutation(k, n_support)[:_K_CL])(keys)
    jj = jnp.arange(_K_PAD)[None, :, None]
    nn = jnp.arange(n_support)[None, None, :]
    idx_pad = jnp.pad(idx, ((0, 0), (0, _K_PAD - _K_CL)), constant_values=-1)
    w0 = (nn == idx_pad[:, :, None]).astype(jnp.float32) * (jj < _K_CL)

    xall = jnp.concatenate(
        [support.reshape(-1, d), query.reshape(-1, d)], axis=0
    )
    emb = _encoder(xall, W, b)
    s_emb = emb[: n_way * n_support].reshape(n_way, n_support, d)
    q_emb = emb[n_way * n_support :].reshape(n_way, n_query, d)

    K = _gram(s_emb)
    u = _kmeans(K, w0)
    p = _proto(u, s_emb)
    return _cdist_logits(q_emb, p)


def kernel(support, query, W, b):
    return _pipeline(support, query, W, b)


# speedup vs baseline: 72.3462x; 2.1146x over previous
import jax
import jax.numpy as jnp
from jax import lax
from jax.experimental import pallas as pl
from jax.experimental.pallas import tpu as pltpu

_K_CL = 5
_K_PAD = 8
_MAX_ITER = 100


def _gram_body(x_ref, w_ref, b_ref, k_ref):
    cb, n, d = x_ref.shape
    x2d = x_ref[...].reshape(cb * n, d)
    emb = (
        jnp.dot(x2d, w_ref[...], preferred_element_type=jnp.float32)
        + b_ref[...]
    )
    for c in range(cb):
        ec = emb[c * n : (c + 1) * n]
        k_ref[c] = lax.dot_general(
            ec, ec, (((1,), (1,)), ((), ())), preferred_element_type=jnp.float32
        )


def _gram(x, W, b, class_block=8):
    n_way, n, d = x.shape
    return pl.pallas_call(
        _gram_body,
        grid=(n_way // class_block,),
        in_specs=[
            pl.BlockSpec((class_block, n, d), lambda i: (i, 0, 0)),
            pl.BlockSpec((d, d), lambda i: (0, 0)),
            pl.BlockSpec((1, d), lambda i: (0, 0)),
        ],
        out_specs=pl.BlockSpec((class_block, n, n), lambda i: (i, 0, 0)),
        out_shape=jax.ShapeDtypeStruct((n_way, n, n), jnp.float32),
    )(x, W, b.reshape(1, d))


def _kmeans_body(k_ref, w0_ref, u_ref, w_ref, prev_ref):
    n_way, kp, n = w0_ref.shape
    w_ref[...] = w0_ref[...]
    prev_ref[...] = jnp.full((n_way, n), -1, jnp.int32)

    jidx = lax.broadcasted_iota(jnp.int32, (kp, n), 0)
    jbad = jidx >= _K_CL

    def one_class(c, carry):
        kc = k_ref[c]

        def cond(it_ch):
            it, changed = it_ch
            return jnp.logical_and(it < _MAX_ITER, changed)

        def body(it_ch):
            it, _ = it_ch
            wc = w_ref[c]
            g = lax.dot_general(
                wc, kc, (((1,), (0,)), ((), ())),
                preferred_element_type=jnp.float32,
            )
            c2 = jnp.sum(g * wc, axis=1, keepdims=True)
            score = c2 - 2.0 * g
            score = jnp.where(jbad, jnp.inf, score)
            smin = jnp.min(score, axis=0, keepdims=True)
            assign = jnp.min(
                jnp.where(score == smin, jidx, _K_PAD), axis=0, keepdims=True
            )
            onehot = (jidx == assign).astype(jnp.float32)
            counts = jnp.sum(onehot, axis=1, keepdims=True)
            w_ref[c] = jnp.where(counts > 0.0, onehot / counts, wc)
            ch = jnp.any(assign[0] != prev_ref[c])
            prev_ref[c] = assign[0]
            return it + 1, ch

        lax.while_loop(cond, body, (jnp.int32(0), jnp.bool_(True)))
        return carry

    lax.fori_loop(0, n_way, one_class, jnp.int32(0))
    u_ref[...] = jnp.sum(w_ref[...], axis=1) * (1.0 / _K_CL)


def _kmeans(K, w0):
    n_way, kp, n = w0.shape
    return pl.pallas_call(
        _kmeans_body,
        in_specs=[
            pl.BlockSpec((n_way, n, n), lambda: (0, 0, 0)),
            pl.BlockSpec((n_way, kp, n), lambda: (0, 0, 0)),
        ],
        out_specs=pl.BlockSpec((n_way, n), lambda: (0, 0)),
        out_shape=jax.ShapeDtypeStruct((n_way, n), jnp.float32),
        scratch_shapes=[
            pltpu.VMEM((n_way, kp, n), jnp.float32),
            pltpu.VMEM((n_way, n), jnp.int32),
        ],
    )(K, w0)


def _proto_body(u_ref, x_ref, w_ref, b_ref, p_ref):
    t = jnp.dot(u_ref[0], x_ref[0], preferred_element_type=jnp.float32)
    p_ref[0] = (
        jnp.dot(t, w_ref[...], preferred_element_type=jnp.float32)
        + b_ref[...]
    )


def _proto(u, x, W, b):
    n_way, n, d = x.shape
    out = pl.pallas_call(
        _proto_body,
        grid=(n_way,),
        in_specs=[
            pl.BlockSpec((1, 1, n), lambda c: (c, 0, 0)),
            pl.BlockSpec((1, n, d), lambda c: (c, 0, 0)),
            pl.BlockSpec((d, d), lambda c: (0, 0)),
            pl.BlockSpec((1, d), lambda c: (0, 0)),
        ],
        out_specs=pl.BlockSpec((1, 1, d), lambda c: (c, 0, 0)),
        out_shape=jax.ShapeDtypeStruct((n_way, 1, d), jnp.float32),
    )(u.reshape(n_way, 1, n), x, W, b.reshape(1, d))
    return out.reshape(n_way, d)


def _cdist_body(q_ref, w_ref, b_ref, p_ref, o_ref):
    q = q_ref[0]
    qe = (
        jnp.dot(q, w_ref[...], preferred_element_type=jnp.float32)
        + b_ref[...]
    )
    p = p_ref[...]
    q2 = jnp.sum(qe * qe, axis=1, keepdims=True)
    p2 = jnp.sum(p * p, axis=1, keepdims=True)
    qp = lax.dot_general(
        qe, p, (((1,), (1,)), ((), ())), preferred_element_type=jnp.float32
    )
    d2 = q2 + p2.T - 2.0 * qp
    o_ref[0] = -jnp.sqrt(jnp.maximum(d2, 1e-12))


def _cdist_logits(q, W, b, p):
    n_way, nq, d = q.shape
    return pl.pallas_call(
        _cdist_body,
        grid=(n_way,),
        in_specs=[
            pl.BlockSpec((1, nq, d), lambda c: (c, 0, 0)),
            pl.BlockSpec((d, d), lambda c: (0, 0)),
            pl.BlockSpec((1, d), lambda c: (0, 0)),
            pl.BlockSpec((n_way, d), lambda c: (0, 0)),
        ],
        out_specs=pl.BlockSpec((1, nq, n_way), lambda c: (c, 0, 0)),
        out_shape=jax.ShapeDtypeStruct((n_way, nq, n_way), jnp.float32),
    )(q, W, b.reshape(1, d), p)


@jax.jit
def _pipeline(support, query, W, b):
    n_way, n_support, d = support.shape

    kkey = jax.random.key(42)
    keys = jax.random.split(kkey, n_way)
    idx = jax.vmap(lambda k: jax.random.permutation(k, n_support)[:_K_CL])(keys)
    jj = jnp.arange(_K_PAD)[None, :, None]
    nn = jnp.arange(n_support)[None, None, :]
    idx_pad = jnp.pad(idx, ((0, 0), (0, _K_PAD - _K_CL)), constant_values=-1)
    w0 = (nn == idx_pad[:, :, None]).astype(jnp.float32) * (jj < _K_CL)

    K = _gram(support, W, b)
    u = _kmeans(K, w0)
    p = _proto(u, support, W, b)
    return _cdist_logits(query, W, b, p)


def kernel(support, query, W, b):
    return _pipeline(support, query, W, b)
